# baseline (speedup 1.0000x reference)
import functools

import jax
import jax.numpy as jnp
from jax import lax
from jax.experimental import pallas as pl
from jax.experimental.pallas import tpu as pltpu
from jax.experimental.pallas import tpu_sc as plsc

B = 16384
D = 64
NC = 2
NS = 16
NW = NC * NS
BPW = B // NW


def _sc_gather(u_ids, i_ids, user_emb, item_emb, ent_emb, item_map):
    mesh = plsc.VectorSubcoreMesh(core_axis_name="c", subcore_axis_name="s")
    out_types = (
        jax.ShapeDtypeStruct((B, D), jnp.float32),
        jax.ShapeDtypeStruct((B, D), jnp.float32),
        jax.ShapeDtypeStruct((B, D), jnp.float32),
    )

    @functools.partial(
        pl.kernel,
        mesh=mesh,
        out_type=out_types,
        compiler_params=pltpu.CompilerParams(use_tc_tiling_on_sc=False),
        scratch_types=[
            pltpu.VMEM((BPW,), jnp.int32),
            pltpu.VMEM((BPW,), jnp.int32),
            pltpu.VMEM((BPW,), jnp.int32),
            pltpu.VMEM((BPW, D), jnp.float32),
            pltpu.VMEM((BPW, D), jnp.float32),
            pltpu.VMEM((BPW, D), jnp.float32),
            pltpu.SemaphoreType.DMA,
            pltpu.SemaphoreType.DMA,
            pltpu.SemaphoreType.DMA,
        ],
    )
    def k(u_ids_h, i_ids_h, ue_h, ie_h, ee_h, map_h, u_out, i_out, e_out,
          uidx, iidx, evar, urows, irows, erows, s0, s1, s2):
        wid = lax.axis_index("s") * NC + lax.axis_index("c")
        base = wid * BPW
        pltpu.sync_copy(u_ids_h.at[pl.ds(base, BPW)], uidx)
        pltpu.sync_copy(i_ids_h.at[pl.ds(base, BPW)], iidx)
        pltpu.async_copy(map_h.at[iidx], evar, s0).wait()
        cu = pltpu.async_copy(ue_h.at[uidx], urows, s0)
        ci = pltpu.async_copy(ie_h.at[iidx], irows, s1)
        ce = pltpu.async_copy(ee_h.at[evar], erows, s2)
        cu.wait()
        pltpu.sync_copy(urows, u_out.at[pl.ds(base, BPW)])
        ci.wait()
        pltpu.sync_copy(irows, i_out.at[pl.ds(base, BPW)])
        ce.wait()
        pltpu.sync_copy(erows, e_out.at[pl.ds(base, BPW)])

    return k(u_ids, i_ids, user_emb, item_emb, ent_emb, item_map)


def _tc_score(u_rows, i_rows, e_rows):
    def body(u_ref, i_ref, e_ref, o_ref):
        o_ref[...] = jnp.sum(u_ref[...] * (i_ref[...] + e_ref[...]), axis=-1)

    return pl.pallas_call(
        body,
        out_shape=jax.ShapeDtypeStruct((B,), jnp.float32),
    )(u_rows, i_rows, e_rows)


def kernel(u_ids, i_ids, user_emb, item_emb, ent_emb, item_map):
    u_rows, i_rows, e_rows = _sc_gather(
        u_ids.astype(jnp.int32), i_ids.astype(jnp.int32),
        user_emb, item_emb, ent_emb, item_map.astype(jnp.int32))
    return _tc_score(u_rows, i_rows, e_rows)
